# baseline (speedup 1.0000x reference)
import functools

import jax
import jax.numpy as jnp
from jax.experimental import pallas as pl
from jax.experimental.pallas import tpu as pltpu

_CH = 10000


def _linear_kernel(x_hbm, w_ref, o_hbm, ibuf, obuf, isem, osem):
    n = x_hbm.shape[0]
    nch = n // _CH

    def in_copy(i):
        return pltpu.make_async_copy(
            x_hbm.at[pl.ds(i * _CH, _CH), :], ibuf.at[i % 2], isem.at[i % 2])

    def out_copy(i):
        return pltpu.make_async_copy(
            obuf.at[i % 2], o_hbm.at[pl.ds(i * _CH, _CH), :], osem.at[i % 2])

    in_copy(0).start()
    if nch > 1:
        in_copy(1).start()
    for i in range(nch):
        in_copy(i).wait()
        if i >= 2:
            out_copy(i - 2).wait()
        obuf[i % 2] = jax.lax.dot_general(
            ibuf[i % 2], w_ref[...],
            dimension_numbers=(((1,), (1,)), ((), ())),
            preferred_element_type=jnp.float32,
        )
        out_copy(i).start()
        if i + 2 < nch:
            in_copy(i + 2).start()
    if nch >= 2:
        out_copy(nch - 2).wait()
    out_copy(nch - 1).wait()


@functools.partial(jax.jit, static_argnames=())
def kernel(x, W):
    n, d_in = x.shape
    d_out = W.shape[0]
    return pl.pallas_call(
        _linear_kernel,
        in_specs=[
            pl.BlockSpec(memory_space=pltpu.MemorySpace.HBM),
            pl.BlockSpec(memory_space=pltpu.MemorySpace.VMEM),
        ],
        out_specs=pl.BlockSpec(memory_space=pltpu.MemorySpace.HBM),
        out_shape=jax.ShapeDtypeStruct((n, d_out), jnp.float32),
        scratch_shapes=[
            pltpu.VMEM((2, _CH, d_in), jnp.float32),
            pltpu.VMEM((2, _CH, d_out), jnp.float32),
            pltpu.SemaphoreType.DMA((2,)),
            pltpu.SemaphoreType.DMA((2,)),
        ],
    )(x, W)
